# baseline (speedup 1.0000x reference)
import functools

import jax
import jax.numpy as jnp
from jax import lax
from jax.experimental import pallas as pl
from jax.experimental.pallas import tpu as pltpu
from jax.experimental.pallas import tpu_sc as plsc

VOCAB = 100000
DIM = 2048
N_HEADS = 16
HEAD_DIM = DIM // N_HEADS
BATCH = 4
SEQ = 4096
TOKENS = BATCH * SEQ
NW = 32
PER_W = TOKENS // NW
W_PER_B = SEQ // PER_W
CHUNK = 16
NCH = PER_W // CHUNK
NBUF = 3


def _sc_gather(tokens, table):
    mesh = plsc.VectorSubcoreMesh(core_axis_name="c", subcore_axis_name="s")

    @functools.partial(
        pl.kernel,
        mesh=mesh,
        out_type=jax.ShapeDtypeStruct((TOKENS, DIM), jnp.float32),
        scratch_types=[
            pltpu.VMEM((NCH, CHUNK), jnp.int32),
            pltpu.VMEM((NBUF, CHUNK, DIM), jnp.float32),
            pltpu.SemaphoreType.DMA,
            pltpu.SemaphoreType.DMA,
            pltpu.SemaphoreType.DMA,
            pltpu.SemaphoreType.DMA,
            pltpu.SemaphoreType.DMA,
            pltpu.SemaphoreType.DMA,
        ],
    )
    def k(idx_hbm, table_hbm, out_hbm, idx_v, rows_v, g0, g1, g2, o0, o1, o2):
        wid = lax.axis_index("s") * 2 + lax.axis_index("c")
        base = wid * PER_W
        pltpu.sync_copy(idx_hbm.at[wid], idx_v)
        gsem, osem = (g0, g1, g2), (o0, o1, o2)

        def start_gather(g):
            b = g % NBUF
            return pltpu.async_copy(
                table_hbm.at[idx_v.at[g]],
                rows_v.at[b], gsem[b])

        def start_out(g):
            b = g % NBUF
            return pltpu.async_copy(
                rows_v.at[b], out_hbm.at[pl.ds(base + g * CHUNK, CHUNK)],
                osem[b])

        gat_cp = [None] * NCH
        out_cp = [None] * NCH
        for g in range(NBUF):
            gat_cp[g] = start_gather(g)
        for g in range(NCH):
            gat_cp[g].wait()
            out_cp[g] = start_out(g)
            p = g - 1
            if p >= 0 and p + NBUF < NCH:
                out_cp[p].wait()
                gat_cp[p + NBUF] = start_gather(p + NBUF)
        for g in range(NCH - NBUF, NCH):
            if g >= 0:
                out_cp[g].wait()

    return k(tokens, table)


ROWB = 512


def _gen_body(cos_ref, sin_ref, mask_ref):
    i = pl.program_id(0)
    t = (lax.broadcasted_iota(jnp.int32, (ROWB, HEAD_DIM), 0) + i * ROWB).astype(
        jnp.float32
    )
    j = lax.broadcasted_iota(jnp.int32, (ROWB, HEAD_DIM), 1)
    half = jnp.where(j < HEAD_DIM // 2, j, j - HEAD_DIM // 2).astype(jnp.float32)
    inv_freq = jnp.exp(half * (-2.0 / HEAD_DIM) * jnp.log(10000.0))
    ang = t * inv_freq
    cos_ref[0] = jnp.cos(ang)
    sin_ref[0] = jnp.sin(ang)
    r = lax.broadcasted_iota(jnp.int32, (ROWB, SEQ), 0) + i * ROWB
    c = lax.broadcasted_iota(jnp.int32, (ROWB, SEQ), 1)
    mask_ref[0, 0] = jnp.where(c > r, -jnp.inf, 0.0).astype(jnp.float32)


def _gen_cos_sin_mask():
    return pl.pallas_call(
        _gen_body,
        grid=(SEQ // ROWB,),
        out_specs=(
            pl.BlockSpec((1, ROWB, HEAD_DIM), lambda i: (0, i, 0)),
            pl.BlockSpec((1, ROWB, HEAD_DIM), lambda i: (0, i, 0)),
            pl.BlockSpec((1, 1, ROWB, SEQ), lambda i: (0, 0, i, 0)),
        ),
        out_shape=(
            jax.ShapeDtypeStruct((1, SEQ, HEAD_DIM), jnp.float32),
            jax.ShapeDtypeStruct((1, SEQ, HEAD_DIM), jnp.float32),
            jax.ShapeDtypeStruct((1, 1, SEQ, SEQ), jnp.float32),
        ),
    )()


def kernel(tokens, W):
    bsz, seq_len = tokens.shape
    cos, sin, mask = _gen_cos_sin_mask()
    idx3 = tokens.reshape(NW, NCH, CHUNK)
    hidden = _sc_gather(idx3, W).reshape(bsz, seq_len, DIM)
    return (hidden, cos, sin, mask)


# speedup vs baseline: 1.0117x; 1.0117x over previous
import functools

import jax
import jax.numpy as jnp
from jax import lax
from jax.experimental import pallas as pl
from jax.experimental.pallas import tpu as pltpu
from jax.experimental.pallas import tpu_sc as plsc

VOCAB = 100000
DIM = 2048
N_HEADS = 16
HEAD_DIM = DIM // N_HEADS
BATCH = 4
SEQ = 4096
TOKENS = BATCH * SEQ
NW = 32
PER_W = TOKENS // NW
W_PER_B = SEQ // PER_W
CHUNK = 16
NCH = PER_W // CHUNK
NBUF = 3


def _sc_gather(tokens, table):
    mesh = plsc.VectorSubcoreMesh(core_axis_name="c", subcore_axis_name="s")

    @functools.partial(
        pl.kernel,
        mesh=mesh,
        out_type=jax.ShapeDtypeStruct((TOKENS, DIM), jnp.float32),
        scratch_types=[
            pltpu.VMEM((PER_W,), jnp.int32),
            pltpu.VMEM((NBUF, CHUNK, DIM), jnp.float32),
            pltpu.SemaphoreType.DMA,
            pltpu.SemaphoreType.DMA,
            pltpu.SemaphoreType.DMA,
            pltpu.SemaphoreType.DMA,
            pltpu.SemaphoreType.DMA,
            pltpu.SemaphoreType.DMA,
        ],
    )
    def k(idx_hbm, table_hbm, out_hbm, idx_v, rows_v, g0, g1, g2, o0, o1, o2):
        wid = lax.axis_index("s") * 2 + lax.axis_index("c")
        base = wid * PER_W
        pltpu.sync_copy(
            idx_hbm.at[wid // W_PER_B, pl.ds((wid % W_PER_B) * PER_W, PER_W)],
            idx_v)
        gsem, osem = (g0, g1, g2), (o0, o1, o2)

        def start_gather(g):
            b = g % NBUF
            return pltpu.async_copy(
                table_hbm.at[idx_v.at[pl.ds(g * CHUNK, CHUNK)]],
                rows_v.at[b], gsem[b])

        def start_out(g):
            b = g % NBUF
            return pltpu.async_copy(
                rows_v.at[b], out_hbm.at[pl.ds(base + g * CHUNK, CHUNK)],
                osem[b])

        gat_cp = [None] * NCH
        out_cp = [None] * NCH
        for g in range(NBUF):
            gat_cp[g] = start_gather(g)
        for g in range(NCH):
            gat_cp[g].wait()
            out_cp[g] = start_out(g)
            p = g - 1
            if p >= 0 and p + NBUF < NCH:
                out_cp[p].wait()
                gat_cp[p + NBUF] = start_gather(p + NBUF)
        for g in range(NCH - NBUF, NCH):
            if g >= 0:
                out_cp[g].wait()

    return k(tokens, table)


ROWB = 512


def _gen_body(cos_ref, sin_ref, mask_ref):
    i = pl.program_id(0)
    t = (lax.broadcasted_iota(jnp.int32, (ROWB, HEAD_DIM), 0) + i * ROWB).astype(
        jnp.float32
    )
    j = lax.broadcasted_iota(jnp.int32, (ROWB, HEAD_DIM), 1)
    half = jnp.where(j < HEAD_DIM // 2, j, j - HEAD_DIM // 2).astype(jnp.float32)
    inv_freq = jnp.exp(half * (-2.0 / HEAD_DIM) * jnp.log(10000.0))
    ang = t * inv_freq
    cos_ref[0] = jnp.cos(ang)
    sin_ref[0] = jnp.sin(ang)
    r = lax.broadcasted_iota(jnp.int32, (ROWB, SEQ), 0) + i * ROWB
    c = lax.broadcasted_iota(jnp.int32, (ROWB, SEQ), 1)
    mask_ref[0, 0] = jnp.where(c > r, -jnp.inf, 0.0).astype(jnp.float32)


def _gen_cos_sin_mask():
    return pl.pallas_call(
        _gen_body,
        grid=(SEQ // ROWB,),
        out_specs=(
            pl.BlockSpec((1, ROWB, HEAD_DIM), lambda i: (0, i, 0)),
            pl.BlockSpec((1, ROWB, HEAD_DIM), lambda i: (0, i, 0)),
            pl.BlockSpec((1, 1, ROWB, SEQ), lambda i: (0, 0, i, 0)),
        ),
        out_shape=(
            jax.ShapeDtypeStruct((1, SEQ, HEAD_DIM), jnp.float32),
            jax.ShapeDtypeStruct((1, SEQ, HEAD_DIM), jnp.float32),
            jax.ShapeDtypeStruct((1, 1, SEQ, SEQ), jnp.float32),
        ),
    )()


def kernel(tokens, W):
    bsz, seq_len = tokens.shape
    cos, sin, mask = _gen_cos_sin_mask()
    hidden = _sc_gather(tokens, W).reshape(bsz, seq_len, DIM)
    return (hidden, cos, sin, mask)


# speedup vs baseline: 1.0126x; 1.0010x over previous
import functools

import jax
import jax.numpy as jnp
from jax import lax
from jax.experimental import pallas as pl
from jax.experimental.pallas import tpu as pltpu
from jax.experimental.pallas import tpu_sc as plsc

VOCAB = 100000
DIM = 2048
N_HEADS = 16
HEAD_DIM = DIM // N_HEADS
BATCH = 4
SEQ = 4096
TOKENS = BATCH * SEQ
NW = 32
PER_W = TOKENS // NW
W_PER_B = SEQ // PER_W
CHUNK = 8
NCH = PER_W // CHUNK
NBUF = 6


def _sc_gather(tokens, table):
    mesh = plsc.VectorSubcoreMesh(core_axis_name="c", subcore_axis_name="s")

    @functools.partial(
        pl.kernel,
        mesh=mesh,
        out_type=jax.ShapeDtypeStruct((TOKENS, DIM), jnp.float32),
        scratch_types=[
            pltpu.VMEM((PER_W,), jnp.int32),
            pltpu.VMEM((NBUF, CHUNK, DIM), jnp.float32),
        ] + [pltpu.SemaphoreType.DMA] * (2 * NBUF),
    )
    def k(idx_hbm, table_hbm, out_hbm, idx_v, rows_v, *sems):
        wid = lax.axis_index("s") * 2 + lax.axis_index("c")
        base = wid * PER_W
        pltpu.sync_copy(
            idx_hbm.at[wid // W_PER_B, pl.ds((wid % W_PER_B) * PER_W, PER_W)],
            idx_v)
        gsem, osem = sems[:NBUF], sems[NBUF:]

        def start_gather(g):
            b = g % NBUF
            return pltpu.async_copy(
                table_hbm.at[idx_v.at[pl.ds(g * CHUNK, CHUNK)]],
                rows_v.at[b], gsem[b])

        def start_out(g):
            b = g % NBUF
            return pltpu.async_copy(
                rows_v.at[b], out_hbm.at[pl.ds(base + g * CHUNK, CHUNK)],
                osem[b])

        gat_cp = [None] * NCH
        out_cp = [None] * NCH
        for g in range(NBUF):
            gat_cp[g] = start_gather(g)
        for g in range(NCH):
            gat_cp[g].wait()
            out_cp[g] = start_out(g)
            p = g - 1
            if p >= 0 and p + NBUF < NCH:
                out_cp[p].wait()
                gat_cp[p + NBUF] = start_gather(p + NBUF)
        for g in range(NCH - NBUF, NCH):
            if g >= 0:
                out_cp[g].wait()

    return k(tokens, table)


ROWB = 512


def _gen_body(cos_ref, sin_ref, mask_ref):
    i = pl.program_id(0)
    t = (lax.broadcasted_iota(jnp.int32, (ROWB, HEAD_DIM), 0) + i * ROWB).astype(
        jnp.float32
    )
    j = lax.broadcasted_iota(jnp.int32, (ROWB, HEAD_DIM), 1)
    half = jnp.where(j < HEAD_DIM // 2, j, j - HEAD_DIM // 2).astype(jnp.float32)
    inv_freq = jnp.exp(half * (-2.0 / HEAD_DIM) * jnp.log(10000.0))
    ang = t * inv_freq
    cos_ref[0] = jnp.cos(ang)
    sin_ref[0] = jnp.sin(ang)
    r = lax.broadcasted_iota(jnp.int32, (ROWB, SEQ), 0) + i * ROWB
    c = lax.broadcasted_iota(jnp.int32, (ROWB, SEQ), 1)
    mask_ref[0, 0] = jnp.where(c > r, -jnp.inf, 0.0).astype(jnp.float32)


def _gen_cos_sin_mask():
    return pl.pallas_call(
        _gen_body,
        grid=(SEQ // ROWB,),
        out_specs=(
            pl.BlockSpec((1, ROWB, HEAD_DIM), lambda i: (0, i, 0)),
            pl.BlockSpec((1, ROWB, HEAD_DIM), lambda i: (0, i, 0)),
            pl.BlockSpec((1, 1, ROWB, SEQ), lambda i: (0, 0, i, 0)),
        ),
        out_shape=(
            jax.ShapeDtypeStruct((1, SEQ, HEAD_DIM), jnp.float32),
            jax.ShapeDtypeStruct((1, SEQ, HEAD_DIM), jnp.float32),
            jax.ShapeDtypeStruct((1, 1, SEQ, SEQ), jnp.float32),
        ),
    )()


def kernel(tokens, W):
    bsz, seq_len = tokens.shape
    cos, sin, mask = _gen_cos_sin_mask()
    hidden = _sc_gather(tokens, W).reshape(bsz, seq_len, DIM)
    return (hidden, cos, sin, mask)
